# baseline (speedup 1.0000x reference)
import functools
import math

import jax
import jax.numpy as jnp
from jax import lax
from jax.experimental import pallas as pl
from jax.experimental.pallas import tpu as pltpu
from jax.experimental.pallas import tpu_sc as plsc

D_MODEL = 64
SCALE = math.sqrt(D_MODEL)

NUM_CORES = 2
NUM_SUBCORES = 16
NW = NUM_CORES * NUM_SUBCORES

B_TOTAL = 16384 * 50
IDX_MINOR = 128
ROWS_PER_CHUNK = 512
IDX_ROWS_PER_CHUNK = ROWS_PER_CHUNK // IDX_MINOR
B_PER_W = B_TOTAL // NW
CHUNKS_PER_W = B_PER_W // ROWS_PER_CHUNK
LANES = 16
VECS_PER_ROW = D_MODEL // LANES


def _body(x_hbm, emb_hbm, out_hbm, idx_v, rows_v, sem):
    wid = lax.axis_index("s") * NUM_CORES + lax.axis_index("c")
    idx_row_base = wid * (B_PER_W // IDX_MINOR)
    out_base = wid * B_PER_W

    def chunk_body(i, carry):
        pltpu.sync_copy(
            x_hbm.at[pl.ds(idx_row_base + i * IDX_ROWS_PER_CHUNK,
                           IDX_ROWS_PER_CHUNK)],
            idx_v,
        )
        copies = []
        for j in range(IDX_ROWS_PER_CHUNK):
            copies.append(
                pltpu.async_copy(
                    emb_hbm.at[idx_v.at[j]],
                    rows_v.at[pl.ds(j * IDX_MINOR, IDX_MINOR)],
                    sem,
                )
            )
        for c in copies:
            c.wait()

        def scale_row(r, carry2):
            for v in range(VECS_PER_ROW):
                rows_v[r, pl.ds(v * LANES, LANES)] = (
                    rows_v[r, pl.ds(v * LANES, LANES)] * SCALE
                )
            return carry2

        lax.fori_loop(0, ROWS_PER_CHUNK, scale_row, 0, unroll=2)

        pltpu.sync_copy(
            rows_v,
            out_hbm.at[pl.ds(out_base + i * ROWS_PER_CHUNK, ROWS_PER_CHUNK)],
        )
        return carry

    lax.fori_loop(0, CHUNKS_PER_W, chunk_body, 0)


@jax.jit
def _gather_scaled(x2d, emb):
    mesh = plsc.VectorSubcoreMesh(core_axis_name="c", subcore_axis_name="s")
    f = pl.kernel(
        _body,
        out_type=jax.ShapeDtypeStruct((B_TOTAL, D_MODEL), jnp.float32),
        mesh=mesh,
        scratch_types=[
            pltpu.VMEM((IDX_ROWS_PER_CHUNK, IDX_MINOR), jnp.int32),
            pltpu.VMEM((ROWS_PER_CHUNK, D_MODEL), jnp.float32),
            pltpu.SemaphoreType.DMA,
        ],
        compiler_params=pltpu.CompilerParams(use_tc_tiling_on_sc=False),
    )
    return f(x2d, emb)


def kernel(x, emb):
    x2d = x.reshape(B_TOTAL // IDX_MINOR, IDX_MINOR).astype(jnp.int32)
    out = _gather_scaled(x2d, emb)
    return out.reshape(x.shape[0], x.shape[1], D_MODEL)
